# speedup vs baseline: 1.1650x; 1.1650x over previous
import jax
import jax.numpy as jnp
from jax.experimental import pallas as pl


def _edge_mlp_body(ga_ref, pd_ref, xv_ref,
                   pw1_ref, pb1_ref, pw2_ref, pb2_ref,
                   aw1_ref, ab1_ref, aw2_ref, ab2_ref,
                   ex_ref, v_ref):
    ga = ga_ref[...]
    pd = pd_ref[...]
    xv = xv_ref[...]
    hp = jax.nn.relu(
        jnp.dot(pd, pw1_ref[...], preferred_element_type=jnp.float32)
        + pb1_ref[...])
    delta = jnp.dot(hp, pw2_ref[...], preferred_element_type=jnp.float32) + pb2_ref[...]
    apre = ga + delta
    hid = jax.nn.relu(
        jnp.dot(apre, aw1_ref[...], preferred_element_type=jnp.float32)
        + ab1_ref[...])
    alpha = jnp.dot(hid, aw2_ref[...], preferred_element_type=jnp.float32) + ab2_ref[...]
    ex = jnp.exp(alpha)
    ex_ref[...] = ex
    v_ref[...] = ex * (xv + delta)


def _edge_mlp(ga, pd, xv, pw1, pb1, pw2, pb2, aw1, ab1, aw2, ab2, block=2048):
    e_tot, c = ga.shape
    assert e_tot % block == 0
    grid = e_tot // block
    hidden = pw1.shape[1]
    row = lambda i: (i, 0)
    w0 = lambda i: (0, 0)
    espec = lambda width: pl.BlockSpec((block, width), row)
    wspec = lambda a, b: pl.BlockSpec((a, b), w0)
    bspec = lambda a: pl.BlockSpec((a,), lambda i: (0,))
    return pl.pallas_call(
        _edge_mlp_body,
        grid=(grid,),
        in_specs=[
            espec(c), espec(2), espec(c),
            wspec(2, hidden), bspec(hidden), wspec(hidden, c), bspec(c),
            wspec(c, hidden), bspec(hidden), wspec(hidden, c), bspec(c),
        ],
        out_specs=[espec(c), espec(c)],
        out_shape=[
            jax.ShapeDtypeStruct((e_tot, c), jnp.float32),
            jax.ShapeDtypeStruct((e_tot, c), jnp.float32),
        ],
    )(ga, pd, xv, pw1, pb1, pw2, pb2, aw1, ab1, aw2, ab2)


def _ptconv(x, pos, src, dst, n, lin, lsrc, ldst, pw1, pb1, pw2, pb2,
            aw1, ab1, aw2, ab2):
    a_src = x @ lsrc
    a_dst = x @ ldst
    xv = x @ lin
    ga = a_dst[dst] - a_src[src]
    pd = pos[dst] - pos[src]
    xvs = xv[src]
    ex, v = _edge_mlp(ga, pd, xvs, pw1, pb1, pw2, pb2, aw1, ab1, aw2, ab2)
    den = jax.ops.segment_sum(ex, dst, num_segments=n)
    num = jax.ops.segment_sum(v, dst, num_segments=n)
    return num / (den + 1e-16)


def kernel(x, pos, edge_index,
           l1_lin, l1_lsrc, l1_ldst, l1_pw1, l1_pb1, l1_pw2, l1_pb2,
           l1_aw1, l1_ab1, l1_aw2, l1_ab2,
           l2_lin, l2_lsrc, l2_ldst, l2_pw1, l2_pb1, l2_pw2, l2_pb2,
           l2_aw1, l2_ab1, l2_aw2, l2_ab2):
    n = x.shape[0]
    e = edge_index.shape[1]
    loops = jnp.arange(n, dtype=edge_index.dtype)
    src = jnp.concatenate([edge_index[0], loops])
    dst = jnp.concatenate([edge_index[1], loops])
    block = 2048
    e_tot = e + n
    pad = (-e_tot) % block
    if pad:
        src = jnp.concatenate([src, jnp.zeros((pad,), src.dtype)])
        dst = jnp.concatenate([dst, jnp.full((pad,), n, dst.dtype)])
    h = _ptconv(x, pos, src, dst, n, l1_lin, l1_lsrc, l1_ldst,
                l1_pw1, l1_pb1, l1_pw2, l1_pb2, l1_aw1, l1_ab1, l1_aw2, l1_ab2)
    h = jax.nn.relu(h)
    out = _ptconv(h, pos, src, dst, n, l2_lin, l2_lsrc, l2_ldst,
                  l2_pw1, l2_pb1, l2_pw2, l2_pb2, l2_aw1, l2_ab1, l2_aw2, l2_ab2)
    return out


# speedup vs baseline: 2.7741x; 2.3811x over previous
import functools

import jax
import jax.numpy as jnp
from jax import lax
from jax.experimental import pallas as pl
from jax.experimental.pallas import tpu as pltpu
from jax.experimental.pallas import tpu_sc as plsc

NC = 2
NS = 16
NW = NC * NS
B = 128


def _round_up(v, m):
    return (v + m - 1) // m * m


def _sc_gather(srcg, dstg, ts, td):
    ep = srcg.shape[0]
    ws = ts.shape[1]
    wd = td.shape[1]
    chunk = ep // NW
    iters = chunk // B
    mesh = plsc.VectorSubcoreMesh(core_axis_name="c", subcore_axis_name="s",
                                  num_cores=NC, num_subcores=NS)

    @functools.partial(
        pl.kernel,
        mesh=mesh,
        out_type=[
            jax.ShapeDtypeStruct((ep, ws), jnp.float32),
            jax.ShapeDtypeStruct((ep, wd), jnp.float32),
        ],
        scratch_types=[
            pltpu.VMEM((B,), jnp.int32),
            pltpu.VMEM((B,), jnp.int32),
            pltpu.VMEM((B, ws), jnp.float32),
            pltpu.VMEM((B, wd), jnp.float32),
            pltpu.SemaphoreType.DMA,
            pltpu.SemaphoreType.DMA,
        ],
    )
    def gk(srcg_h, dstg_h, ts_h, td_h, outs_h, outd_h,
           sidx, didx, sbuf, dbuf, sem1, sem2):
        wid = lax.axis_index("s") * NC + lax.axis_index("c")
        base = wid * chunk

        def body(i, carry):
            off = base + i * B
            pltpu.sync_copy(srcg_h.at[pl.ds(off, B)], sidx)
            pltpu.async_copy(ts_h.at[sidx], sbuf, sem1).wait()
            pltpu.sync_copy(sbuf, outs_h.at[pl.ds(off, B)])
            pltpu.sync_copy(dstg_h.at[pl.ds(off, B)], didx)
            pltpu.async_copy(td_h.at[didx], dbuf, sem2).wait()
            pltpu.sync_copy(dbuf, outd_h.at[pl.ds(off, B)])
            return carry

        lax.fori_loop(0, iters, body, 0)

    return gk(srcg, dstg, ts, td)


def _sc_scatter(rows, dsts, nr):
    return jnp.stack([
        jax.ops.segment_sum(rows[c], dsts[c], num_segments=nr)
        for c in range(rows.shape[0])])


def _sc_scatter_real(rows, dsts, nr):
    _, ec, w = rows.shape
    per_sub = ec // NS
    iters = per_sub // B
    zrows = nr // NS
    ziters = zrows // B
    mesh = plsc.VectorSubcoreMesh(core_axis_name="c", subcore_axis_name="s",
                                  num_cores=NC, num_subcores=NS)

    @functools.partial(
        pl.kernel,
        mesh=mesh,
        out_type=jax.ShapeDtypeStruct((NC, nr, w), jnp.float32),
        scratch_types=[
            pltpu.VMEM((B,), jnp.int32),
            pltpu.VMEM((B, w), jnp.float32),
            pltpu.VMEM((B, w), jnp.float32),
            pltpu.VMEM_SHARED((nr, w), jnp.float32),
            pltpu.SemaphoreType.DMA,
        ],
    )
    def sk(rows_h, dsts_h, acc_h, idxv, rbuf, zbuf, table, sem):
        c = lax.axis_index("c")
        s = lax.axis_index("s")

        def zb(i, carry):
            r = i // (w // 16)
            col = (i % (w // 16)) * 16
            zbuf[r, pl.ds(col, 16)] = jnp.zeros((16,), jnp.float32)
            return carry

        lax.fori_loop(0, B * (w // 16), zb, 0)

        def zt(k, carry):
            pltpu.sync_copy(zbuf, table.at[pl.ds(s * zrows + k * B, B)])
            return carry

        lax.fori_loop(0, ziters, zt, 0)
        plsc.subcore_barrier()

        base = s * per_sub

        def body(i, carry):
            off = base + i * B
            pltpu.sync_copy(dsts_h.at[c, pl.ds(off, B)], idxv)
            pltpu.sync_copy(rows_h.at[c, pl.ds(off, B)], rbuf)
            pltpu.sync_copy(rbuf, table.at[idxv], add=True)
            return carry

        lax.fori_loop(0, iters, body, 0)
        plsc.subcore_barrier()

        def wb(k, carry):
            off = s * zrows + k * B
            pltpu.sync_copy(table.at[pl.ds(off, B)], zbuf)
            pltpu.sync_copy(zbuf, acc_h.at[c, pl.ds(off, B)])
            return carry

        lax.fori_loop(0, ziters, wb, 0)

    return sk(rows, dsts)


def _prep_body(x_ref, pos_ref, lin_ref, lsrc_ref, ldst_ref, ts_ref, td_ref):
    x = x_ref[...]
    pos = pos_ref[...]
    nb = x.shape[0]
    c = lin_ref.shape[1]
    xv = jnp.dot(x, lin_ref[...], preferred_element_type=jnp.float32)
    a_s = jnp.dot(x, lsrc_ref[...], preferred_element_type=jnp.float32)
    a_d = jnp.dot(x, ldst_ref[...], preferred_element_type=jnp.float32)
    pad_s = jnp.zeros((nb, ts_ref.shape[1] - 2 * c - 2), jnp.float32)
    pad_d = jnp.zeros((nb, td_ref.shape[1] - c - 2), jnp.float32)
    ts_ref[...] = jnp.concatenate([xv, a_s, pos, pad_s], axis=1)
    td_ref[...] = jnp.concatenate([a_d, pos, pad_d], axis=1)


def _prep(x, pos, lin, lsrc, ldst, ws, wd, nb=400):
    n, cin = x.shape
    c = lin.shape[1]
    grid = n // nb
    row = lambda i: (i, 0)
    w0 = lambda i: (0, 0)
    return pl.pallas_call(
        _prep_body,
        grid=(grid,),
        in_specs=[
            pl.BlockSpec((nb, cin), row),
            pl.BlockSpec((nb, 2), row),
            pl.BlockSpec((cin, c), w0),
            pl.BlockSpec((cin, c), w0),
            pl.BlockSpec((cin, c), w0),
        ],
        out_specs=[
            pl.BlockSpec((nb, ws), row),
            pl.BlockSpec((nb, wd), row),
        ],
        out_shape=[
            jax.ShapeDtypeStruct((n, ws), jnp.float32),
            jax.ShapeDtypeStruct((n, wd), jnp.float32),
        ],
    )(x, pos, lin, lsrc, ldst)


def _mid_body(c, split, sg_ref, dg_ref,
              pw1_ref, pb1_ref, pw2_ref, pb2_ref,
              aw1_ref, ab1_ref, aw2_ref, ab2_ref, *out_ref):
    sg = sg_ref[...]
    dg = dg_ref[...]
    xv = sg[:, 0:c]
    a_s = sg[:, c:2 * c]
    pos_s = sg[:, 2 * c:2 * c + 2]
    a_d = dg[:, 0:c]
    pos_d = dg[:, c:c + 2]
    pd = pos_d - pos_s
    hp = jax.nn.relu(
        jnp.dot(pd, pw1_ref[...], preferred_element_type=jnp.float32)
        + pb1_ref[...])
    delta = jnp.dot(hp, pw2_ref[...], preferred_element_type=jnp.float32) + pb2_ref[...]
    apre = a_d - a_s + delta
    hid = jax.nn.relu(
        jnp.dot(apre, aw1_ref[...], preferred_element_type=jnp.float32)
        + ab1_ref[...])
    alpha = jnp.dot(hid, aw2_ref[...], preferred_element_type=jnp.float32) + ab2_ref[...]
    ex = jnp.exp(alpha)
    v = ex * (xv + delta)
    if split:
        h = c // 2
        vout_ref, eout_ref = out_ref
        vout_ref[0] = v[:, 0:h]
        vout_ref[1] = v[:, h:c]
        eout_ref[0] = ex[:, 0:h]
        eout_ref[1] = ex[:, h:c]
    else:
        out_ref[0][...] = jnp.concatenate([v, ex], axis=1)


def _mid(c, split, sg, dg, pw1, pb1, pw2, pb2, aw1, ab1, aw2, ab2, be=2048):
    ep, ws = sg.shape
    wd = dg.shape[1]
    hidden = pw1.shape[1]
    grid = ep // be
    row = lambda i: (i, 0)
    w0 = lambda i: (0, 0)
    b0 = lambda i: (0,)
    if split:
        h = c // 2
        out_spec = [pl.BlockSpec((NC, be, h), lambda i: (0, i, 0)),
                    pl.BlockSpec((NC, be, h), lambda i: (0, i, 0))]
        out_shape = [jax.ShapeDtypeStruct((NC, ep, h), jnp.float32),
                     jax.ShapeDtypeStruct((NC, ep, h), jnp.float32)]
    else:
        out_spec = [pl.BlockSpec((be, 2 * c), row)]
        out_shape = [jax.ShapeDtypeStruct((ep, 2 * c), jnp.float32)]
    return pl.pallas_call(
        functools.partial(_mid_body, c, split),
        grid=(grid,),
        in_specs=[
            pl.BlockSpec((be, ws), row),
            pl.BlockSpec((be, wd), row),
            pl.BlockSpec((2, hidden), w0), pl.BlockSpec((hidden,), b0),
            pl.BlockSpec((hidden, c), w0), pl.BlockSpec((c,), b0),
            pl.BlockSpec((c, hidden), w0), pl.BlockSpec((hidden,), b0),
            pl.BlockSpec((hidden, c), w0), pl.BlockSpec((c,), b0),
        ],
        out_specs=out_spec,
        out_shape=out_shape,
    )(sg, dg, pw1, pb1, pw2, pb2, aw1, ab1, aw2, ab2)


def _selfloop_terms(a_d, a_s, xv, pw1b, pw2, pb2, aw1, ab1, aw2, ab2):
    hp0 = jax.nn.relu(pw1b)
    delta0 = jnp.dot(hp0.reshape(1, -1), pw2,
                     preferred_element_type=jnp.float32) + pb2
    apre = a_d - a_s + delta0
    hid = jax.nn.relu(
        jnp.dot(apre, aw1, preferred_element_type=jnp.float32) + ab1)
    alpha = jnp.dot(hid, aw2, preferred_element_type=jnp.float32) + ab2
    ex = jnp.exp(alpha)
    return ex, ex * (xv + delta0)


def _final1_body(c, accv_ref, acce_ref, ts_ref, td_ref,
                 pb1_ref, pw2_ref, pb2_ref, aw1_ref, ab1_ref, aw2_ref, ab2_ref,
                 lin2_ref, lsrc2_ref, ldst2_ref, t2s_ref, t2d_ref):
    num = jnp.concatenate([accv_ref[0], accv_ref[1]], axis=1)
    den = jnp.concatenate([acce_ref[0], acce_ref[1]], axis=1)
    sg = ts_ref[...]
    dg = td_ref[...]
    xv = sg[:, 0:c]
    a_s = sg[:, c:2 * c]
    pos = sg[:, 2 * c:2 * c + 2]
    a_d = dg[:, 0:c]
    ex_sl, v_sl = _selfloop_terms(
        a_d, a_s, xv, pb1_ref[...], pw2_ref[...], pb2_ref[...],
        aw1_ref[...], ab1_ref[...], aw2_ref[...], ab2_ref[...])
    hmat = jax.nn.relu((num + v_sl) / (den + ex_sl + 1e-16))
    nb = hmat.shape[0]
    c2 = lin2_ref.shape[1]
    xv2 = jnp.dot(hmat, lin2_ref[...], preferred_element_type=jnp.float32)
    a_s2 = jnp.dot(hmat, lsrc2_ref[...], preferred_element_type=jnp.float32)
    a_d2 = jnp.dot(hmat, ldst2_ref[...], preferred_element_type=jnp.float32)
    pad_s = jnp.zeros((nb, t2s_ref.shape[1] - 2 * c2 - 2), jnp.float32)
    pad_d = jnp.zeros((nb, t2d_ref.shape[1] - c2 - 2), jnp.float32)
    t2s_ref[...] = jnp.concatenate([xv2, a_s2, pos, pad_s], axis=1)
    t2d_ref[...] = jnp.concatenate([a_d2, pos, pad_d], axis=1)


def _final1(c, accv, acce, ts, td, pb1, pw2, pb2, aw1, ab1, aw2, ab2,
            lin2, lsrc2, ldst2, ws2, wd2, nb=400):
    n = ts.shape[0]
    ws = ts.shape[1]
    wd = td.shape[1]
    hidden = pw2.shape[0]
    c2 = lin2.shape[1]
    grid = n // nb
    row = lambda i: (i, 0)
    w0 = lambda i: (0, 0)
    b0 = lambda i: (0,)
    return pl.pallas_call(
        functools.partial(_final1_body, c),
        grid=(grid,),
        in_specs=[
            pl.BlockSpec((NC, nb, c // 2), lambda i: (0, i, 0)),
            pl.BlockSpec((NC, nb, c // 2), lambda i: (0, i, 0)),
            pl.BlockSpec((nb, ws), row),
            pl.BlockSpec((nb, wd), row),
            pl.BlockSpec((hidden,), b0),
            pl.BlockSpec((hidden, c), w0), pl.BlockSpec((c,), b0),
            pl.BlockSpec((c, hidden), w0), pl.BlockSpec((hidden,), b0),
            pl.BlockSpec((hidden, c), w0), pl.BlockSpec((c,), b0),
            pl.BlockSpec((c, c2), w0), pl.BlockSpec((c, c2), w0),
            pl.BlockSpec((c, c2), w0),
        ],
        out_specs=[
            pl.BlockSpec((nb, ws2), row),
            pl.BlockSpec((nb, wd2), row),
        ],
        out_shape=[
            jax.ShapeDtypeStruct((n, ws2), jnp.float32),
            jax.ShapeDtypeStruct((n, wd2), jnp.float32),
        ],
    )(accv, acce, ts, td, pb1, pw2, pb2, aw1, ab1, aw2, ab2,
      lin2, lsrc2, ldst2)


def _final2_body(c, acc_ref, ts_ref, td_ref,
                 pb1_ref, pw2_ref, pb2_ref, aw1_ref, ab1_ref, aw2_ref, ab2_ref,
                 out_ref):
    accs = acc_ref[0] + acc_ref[1]
    num = accs[:, 0:c]
    den = accs[:, c:2 * c]
    sg = ts_ref[...]
    dg = td_ref[...]
    xv = sg[:, 0:c]
    a_s = sg[:, c:2 * c]
    a_d = dg[:, 0:c]
    ex_sl, v_sl = _selfloop_terms(
        a_d, a_s, xv, pb1_ref[...], pw2_ref[...], pb2_ref[...],
        aw1_ref[...], ab1_ref[...], aw2_ref[...], ab2_ref[...])
    out_ref[...] = (num + v_sl) / (den + ex_sl + 1e-16)


def _final2(c, acc, ts, td, pb1, pw2, pb2, aw1, ab1, aw2, ab2, nb=400):
    n = ts.shape[0]
    ws = ts.shape[1]
    wd = td.shape[1]
    hidden = pw2.shape[0]
    grid = n // nb
    row = lambda i: (i, 0)
    w0 = lambda i: (0, 0)
    b0 = lambda i: (0,)
    return pl.pallas_call(
        functools.partial(_final2_body, c),
        grid=(grid,),
        in_specs=[
            pl.BlockSpec((NC, nb, 2 * c), lambda i: (0, i, 0)),
            pl.BlockSpec((nb, ws), row),
            pl.BlockSpec((nb, wd), row),
            pl.BlockSpec((hidden,), b0),
            pl.BlockSpec((hidden, c), w0), pl.BlockSpec((c,), b0),
            pl.BlockSpec((c, hidden), w0), pl.BlockSpec((hidden,), b0),
            pl.BlockSpec((hidden, c), w0), pl.BlockSpec((c,), b0),
        ],
        out_specs=[pl.BlockSpec((nb, c), row)],
        out_shape=[jax.ShapeDtypeStruct((n, c), jnp.float32)],
    )(acc, ts, td, pb1, pw2, pb2, aw1, ab1, aw2, ab2)[0]


def kernel(x, pos, edge_index,
           l1_lin, l1_lsrc, l1_ldst, l1_pw1, l1_pb1, l1_pw2, l1_pb2,
           l1_aw1, l1_ab1, l1_aw2, l1_ab2,
           l2_lin, l2_lsrc, l2_ldst, l2_pw1, l2_pb1, l2_pw2, l2_pb2,
           l2_aw1, l2_ab1, l2_aw2, l2_ab2):
    n = x.shape[0]
    e = edge_index.shape[1]
    ep = _round_up(e, NW * B * 2)
    nr = _round_up(n + 8, NS * B)
    pad = ep - e

    src = edge_index[0]
    dst = edge_index[1]
    zpad = jnp.zeros((pad,), jnp.int32)
    srcg = jnp.concatenate([src, zpad])
    dstg = jnp.concatenate([dst, zpad])
    dump = n + (jnp.arange(pad, dtype=jnp.int32) % 8)
    dsts = jnp.concatenate([dst, dump])
    dsts1 = jnp.stack([dsts, dsts])
    dsts2 = dsts.reshape(NC, ep // NC)

    c1, c2 = l1_lin.shape[1], l2_lin.shape[1]
    ws1, wd1 = 128, 128
    ws2, wd2 = 128, 128

    t1s, t1d = _prep(x, pos, l1_lin, l1_lsrc, l1_ldst, ws1, wd1)
    sg1, dg1 = _sc_gather(srcg, dstg, t1s, t1d)
    rows1v, rows1e = _mid(c1, True, sg1, dg1, l1_pw1, l1_pb1, l1_pw2, l1_pb2,
                          l1_aw1, l1_ab1, l1_aw2, l1_ab2)
    acc1v = _sc_scatter(rows1v, dsts1, nr)
    acc1e = _sc_scatter(rows1e, dsts1, nr)
    t2s, t2d = _final1(c1, acc1v, acc1e, t1s, t1d, l1_pb1, l1_pw2, l1_pb2,
                       l1_aw1, l1_ab1, l1_aw2, l1_ab2,
                       l2_lin, l2_lsrc, l2_ldst, ws2, wd2)

    sg2, dg2 = _sc_gather(srcg, dstg, t2s, t2d)
    rows2 = _mid(c2, False, sg2, dg2, l2_pw1, l2_pb1, l2_pw2, l2_pb2,
                 l2_aw1, l2_ab1, l2_aw2, l2_ab2)[0]
    rows2 = rows2.reshape(NC, ep // NC, 2 * c2)
    acc2 = _sc_scatter(rows2, dsts2, nr)
    out = _final2(c2, acc2, t2s, t2d, l2_pb1, l2_pw2, l2_pb2,
                  l2_aw1, l2_ab1, l2_aw2, l2_ab2)
    return out


# speedup vs baseline: 4.8617x; 1.7525x over previous
import functools

import jax
import jax.numpy as jnp
from jax import lax
from jax.experimental import pallas as pl
from jax.experimental.pallas import tpu as pltpu
from jax.experimental.pallas import tpu_sc as plsc

NC = 2
NS = 16
NW = NC * NS
B = 128
TW = 128


def _round_up(v, m):
    return (v + m - 1) // m * m


def _sc_gather(srcg, dstg, ts, td):
    ep = srcg.shape[0]
    chunk = ep // NW
    iters = chunk // B
    mesh = plsc.VectorSubcoreMesh(core_axis_name="c", subcore_axis_name="s",
                                  num_cores=NC, num_subcores=NS)

    @functools.partial(
        pl.kernel,
        mesh=mesh,
        out_type=[
            jax.ShapeDtypeStruct((ep, TW), jnp.float32),
            jax.ShapeDtypeStruct((ep, TW), jnp.float32),
        ],
        scratch_types=[
            pltpu.VMEM((B,), jnp.int32),
            pltpu.VMEM((B,), jnp.int32),
            pltpu.VMEM((B, TW), jnp.float32),
            pltpu.VMEM((B, TW), jnp.float32),
            pltpu.SemaphoreType.DMA,
            pltpu.SemaphoreType.DMA,
        ],
    )
    def gk(srcg_h, dstg_h, ts_h, td_h, outs_h, outd_h,
           sidx, didx, sbuf, dbuf, sem1, sem2):
        wid = lax.axis_index("s") * NC + lax.axis_index("c")
        base = wid * chunk

        def body(i, carry):
            off = base + i * B
            pltpu.sync_copy(srcg_h.at[pl.ds(off, B)], sidx)
            pltpu.async_copy(ts_h.at[sidx], sbuf, sem1).wait()
            pltpu.sync_copy(sbuf, outs_h.at[pl.ds(off, B)])
            pltpu.sync_copy(dstg_h.at[pl.ds(off, B)], didx)
            pltpu.async_copy(td_h.at[didx], dbuf, sem2).wait()
            pltpu.sync_copy(dbuf, outd_h.at[pl.ds(off, B)])
            return carry

        lax.fori_loop(0, iters, body, 0)

    return gk(srcg, dstg, ts, td)


def _segsum(rows, dsts, nr):
    return jax.ops.segment_sum(rows, dsts, num_segments=nr)


def _prep_body(x_ref, pos_ref, lin_ref, lsrc_ref, ldst_ref, ts_ref, td_ref):
    x = x_ref[...]
    pos = pos_ref[...]
    nb = x.shape[0]
    c = lin_ref.shape[1]
    xv = jnp.dot(x, lin_ref[...], preferred_element_type=jnp.float32)
    a_s = jnp.dot(x, lsrc_ref[...], preferred_element_type=jnp.float32)
    a_d = jnp.dot(x, ldst_ref[...], preferred_element_type=jnp.float32)
    pad_s = jnp.zeros((nb, TW - 2 * c - 2), jnp.float32)
    pad_d = jnp.zeros((nb, TW - c - 2), jnp.float32)
    ts_ref[...] = jnp.concatenate([xv, a_s, pos, pad_s], axis=1)
    td_ref[...] = jnp.concatenate([a_d, pos, pad_d], axis=1)


def _prep(x, pos, lin, lsrc, ldst, nb=400):
    n, cin = x.shape
    c = lin.shape[1]
    grid = n // nb
    row = lambda i: (i, 0)
    w0 = lambda i: (0, 0)
    return pl.pallas_call(
        _prep_body,
        grid=(grid,),
        in_specs=[
            pl.BlockSpec((nb, cin), row),
            pl.BlockSpec((nb, 2), row),
            pl.BlockSpec((cin, c), w0),
            pl.BlockSpec((cin, c), w0),
            pl.BlockSpec((cin, c), w0),
        ],
        out_specs=[
            pl.BlockSpec((nb, TW), row),
            pl.BlockSpec((nb, TW), row),
        ],
        out_shape=[
            jax.ShapeDtypeStruct((n, TW), jnp.float32),
            jax.ShapeDtypeStruct((n, TW), jnp.float32),
        ],
    )(x, pos, lin, lsrc, ldst)


def _mid_body(c, doff, sg_ref, dg_ref,
              pw1_ref, pb1_ref, pw2_ref, pb2_ref,
              aw1_ref, ab1_ref, aw2_ref, ab2_ref, out_ref):
    sg = sg_ref[...]
    dg = dg_ref[...]
    xv = sg[:, 0:c]
    a_s = sg[:, c:2 * c]
    pos_s = sg[:, 2 * c:2 * c + 2]
    a_d = dg[:, doff:doff + c]
    pos_d = dg[:, doff + c:doff + c + 2]
    pd = pos_d - pos_s
    hp = jax.nn.relu(
        jnp.dot(pd, pw1_ref[...], preferred_element_type=jnp.float32)
        + pb1_ref[...])
    delta = jnp.dot(hp, pw2_ref[...], preferred_element_type=jnp.float32) + pb2_ref[...]
    apre = a_d - a_s + delta
    hid = jax.nn.relu(
        jnp.dot(apre, aw1_ref[...], preferred_element_type=jnp.float32)
        + ab1_ref[...])
    alpha = jnp.dot(hid, aw2_ref[...], preferred_element_type=jnp.float32) + ab2_ref[...]
    ex = jnp.exp(alpha)
    v = ex * (xv + delta)
    out_ref[...] = jnp.concatenate([v, ex], axis=1)


def _mid(c, doff, sg, dg, pw1, pb1, pw2, pb2, aw1, ab1, aw2, ab2, be=2048):
    ep, sw = sg.shape
    dw = dg.shape[1]
    hidden = pw1.shape[1]
    grid = ep // be
    row = lambda i: (i, 0)
    w0 = lambda i: (0, 0)
    b0 = lambda i: (0,)
    return pl.pallas_call(
        functools.partial(_mid_body, c, doff),
        grid=(grid,),
        in_specs=[
            pl.BlockSpec((be, sw), row),
            pl.BlockSpec((be, dw), row),
            pl.BlockSpec((2, hidden), w0), pl.BlockSpec((hidden,), b0),
            pl.BlockSpec((hidden, c), w0), pl.BlockSpec((c,), b0),
            pl.BlockSpec((c, hidden), w0), pl.BlockSpec((hidden,), b0),
            pl.BlockSpec((hidden, c), w0), pl.BlockSpec((c,), b0),
        ],
        out_specs=[pl.BlockSpec((be, 2 * c), row)],
        out_shape=[jax.ShapeDtypeStruct((ep, 2 * c), jnp.float32)],
    )(sg, dg, pw1, pb1, pw2, pb2, aw1, ab1, aw2, ab2)[0]


def _selfloop_terms(a_d, a_s, xv, pw1b, pw2, pb2, aw1, ab1, aw2, ab2):
    hp0 = jax.nn.relu(pw1b)
    delta0 = jnp.dot(hp0.reshape(1, -1), pw2,
                     preferred_element_type=jnp.float32) + pb2
    apre = a_d - a_s + delta0
    hid = jax.nn.relu(
        jnp.dot(apre, aw1, preferred_element_type=jnp.float32) + ab1)
    alpha = jnp.dot(hid, aw2, preferred_element_type=jnp.float32) + ab2
    ex = jnp.exp(alpha)
    return ex, ex * (xv + delta0)


def _final1_body(c, acc_ref, ts_ref, td_ref,
                 pb1_ref, pw2_ref, pb2_ref, aw1_ref, ab1_ref, aw2_ref, ab2_ref,
                 lin2_ref, lsrc2_ref, ldst2_ref, t2_ref):
    num = acc_ref[:, 0:c]
    den = acc_ref[:, c:2 * c]
    sg = ts_ref[...]
    dg = td_ref[...]
    xv = sg[:, 0:c]
    a_s = sg[:, c:2 * c]
    pos = sg[:, 2 * c:2 * c + 2]
    a_d = dg[:, 0:c]
    ex_sl, v_sl = _selfloop_terms(
        a_d, a_s, xv, pb1_ref[...], pw2_ref[...], pb2_ref[...],
        aw1_ref[...], ab1_ref[...], aw2_ref[...], ab2_ref[...])
    hmat = jax.nn.relu((num + v_sl) / (den + ex_sl + 1e-16))
    nb = hmat.shape[0]
    c2 = lin2_ref.shape[1]
    xv2 = jnp.dot(hmat, lin2_ref[...], preferred_element_type=jnp.float32)
    a_s2 = jnp.dot(hmat, lsrc2_ref[...], preferred_element_type=jnp.float32)
    a_d2 = jnp.dot(hmat, ldst2_ref[...], preferred_element_type=jnp.float32)
    z6 = jnp.zeros((nb, 6), jnp.float32)
    zrest = jnp.zeros((nb, TW - 34), jnp.float32)
    t2_ref[...] = jnp.concatenate(
        [xv2, a_s2, pos, z6, a_d2, pos, zrest], axis=1)


def _final1(c, acc, ts, td, pb1, pw2, pb2, aw1, ab1, aw2, ab2,
            lin2, lsrc2, ldst2, nb=400):
    n = ts.shape[0]
    hidden = pw2.shape[0]
    c2 = lin2.shape[1]
    grid = n // nb
    row = lambda i: (i, 0)
    w0 = lambda i: (0, 0)
    b0 = lambda i: (0,)
    return pl.pallas_call(
        functools.partial(_final1_body, c),
        grid=(grid,),
        in_specs=[
            pl.BlockSpec((nb, 2 * c), row),
            pl.BlockSpec((nb, TW), row),
            pl.BlockSpec((nb, TW), row),
            pl.BlockSpec((hidden,), b0),
            pl.BlockSpec((hidden, c), w0), pl.BlockSpec((c,), b0),
            pl.BlockSpec((c, hidden), w0), pl.BlockSpec((hidden,), b0),
            pl.BlockSpec((hidden, c), w0), pl.BlockSpec((c,), b0),
            pl.BlockSpec((c, c2), w0), pl.BlockSpec((c, c2), w0),
            pl.BlockSpec((c, c2), w0),
        ],
        out_specs=[pl.BlockSpec((nb, TW), row)],
        out_shape=[jax.ShapeDtypeStruct((n, TW), jnp.float32)],
    )(acc, ts, td, pb1, pw2, pb2, aw1, ab1, aw2, ab2,
      lin2, lsrc2, ldst2)[0]


def _final2_body(c, acc_ref, t2_ref,
                 pb1_ref, pw2_ref, pb2_ref, aw1_ref, ab1_ref, aw2_ref, ab2_ref,
                 out_ref):
    num = acc_ref[:, 0:c]
    den = acc_ref[:, c:2 * c]
    t2 = t2_ref[...]
    xv = t2[:, 0:c]
    a_s = t2[:, c:2 * c]
    a_d = t2[:, 24:24 + c]
    ex_sl, v_sl = _selfloop_terms(
        a_d, a_s, xv, pb1_ref[...], pw2_ref[...], pb2_ref[...],
        aw1_ref[...], ab1_ref[...], aw2_ref[...], ab2_ref[...])
    out_ref[...] = (num + v_sl) / (den + ex_sl + 1e-16)


def _final2(c, acc, t2, pb1, pw2, pb2, aw1, ab1, aw2, ab2, nb=400):
    n = t2.shape[0]
    hidden = pw2.shape[0]
    grid = n // nb
    row = lambda i: (i, 0)
    w0 = lambda i: (0, 0)
    b0 = lambda i: (0,)
    return pl.pallas_call(
        functools.partial(_final2_body, c),
        grid=(grid,),
        in_specs=[
            pl.BlockSpec((nb, 2 * c), row),
            pl.BlockSpec((nb, TW), row),
            pl.BlockSpec((hidden,), b0),
            pl.BlockSpec((hidden, c), w0), pl.BlockSpec((c,), b0),
            pl.BlockSpec((c, hidden), w0), pl.BlockSpec((hidden,), b0),
            pl.BlockSpec((hidden, c), w0), pl.BlockSpec((c,), b0),
        ],
        out_specs=[pl.BlockSpec((nb, c), row)],
        out_shape=[jax.ShapeDtypeStruct((n, c), jnp.float32)],
    )(acc, t2, pb1, pw2, pb2, aw1, ab1, aw2, ab2)[0]


def kernel(x, pos, edge_index,
           l1_lin, l1_lsrc, l1_ldst, l1_pw1, l1_pb1, l1_pw2, l1_pb2,
           l1_aw1, l1_ab1, l1_aw2, l1_ab2,
           l2_lin, l2_lsrc, l2_ldst, l2_pw1, l2_pb1, l2_pw2, l2_pb2,
           l2_aw1, l2_ab1, l2_aw2, l2_ab2):
    n = x.shape[0]
    e = edge_index.shape[1]
    ep = _round_up(e, NW * B)
    pad = ep - e

    src = edge_index[0]
    dst = edge_index[1]
    zpad = jnp.zeros((pad,), jnp.int32)
    srcg = jnp.concatenate([src, zpad])
    dstg = jnp.concatenate([dst, zpad])
    dsts = jnp.concatenate([dst, jnp.full((pad,), n, jnp.int32)])

    c1, c2 = l1_lin.shape[1], l2_lin.shape[1]

    t1s, t1d = _prep(x, pos, l1_lin, l1_lsrc, l1_ldst)
    sg1, dg1 = _sc_gather(srcg, dstg, t1s, t1d)
    rows1 = _mid(c1, 0, sg1, dg1, l1_pw1, l1_pb1, l1_pw2, l1_pb2,
                 l1_aw1, l1_ab1, l1_aw2, l1_ab2)
    acc1 = _segsum(rows1, dsts, n)
    t2 = _final1(c1, acc1, t1s, t1d, l1_pb1, l1_pw2, l1_pb2,
                 l1_aw1, l1_ab1, l1_aw2, l1_ab2,
                 l2_lin, l2_lsrc, l2_ldst)

    sg2, dg2 = _sc_gather(srcg, dstg, t2, t2)
    rows2 = _mid(c2, 24, sg2, dg2, l2_pw1, l2_pb1, l2_pw2, l2_pb2,
                 l2_aw1, l2_ab1, l2_aw2, l2_ab2)
    acc2 = _segsum(rows2, dsts, n)
    out = _final2(c2, acc2, t2, l2_pb1, l2_pw2, l2_pb2,
                  l2_aw1, l2_ab1, l2_aw2, l2_ab2)
    return out


# speedup vs baseline: 5.4150x; 1.1138x over previous
import functools

import jax
import jax.numpy as jnp
from jax import lax
from jax.experimental import pallas as pl
from jax.experimental.pallas import tpu as pltpu
from jax.experimental.pallas import tpu_sc as plsc

NC = 2
NS = 16
NW = NC * NS
B = 128
TW = 128


def _round_up(v, m):
    return (v + m - 1) // m * m


def _sc_gather(srcg, dstg, ts, td, packs):
    ep = srcg.shape[0]
    chunk = ep // NW
    iters = chunk // B
    mesh = plsc.VectorSubcoreMesh(core_axis_name="c", subcore_axis_name="s",
                                  num_cores=NC, num_subcores=NS)

    @functools.partial(
        pl.kernel,
        mesh=mesh,
        out_type=jax.ShapeDtypeStruct((ep, TW), jnp.float32),
        scratch_types=[
            pltpu.VMEM((B,), jnp.int32),
            pltpu.VMEM((B,), jnp.int32),
            pltpu.VMEM((B, TW), jnp.float32),
            pltpu.VMEM((B, TW), jnp.float32),
            pltpu.SemaphoreType.DMA,
            pltpu.SemaphoreType.DMA,
        ],
    )
    def gk(srcg_h, dstg_h, ts_h, td_h, out_h,
           sidx, didx, sbuf, dbuf, sem1, sem2):
        wid = lax.axis_index("s") * NC + lax.axis_index("c")
        base = wid * chunk

        def body(i, carry):
            off = base + i * B
            pltpu.sync_copy(srcg_h.at[pl.ds(off, B)], sidx)
            pltpu.sync_copy(dstg_h.at[pl.ds(off, B)], didx)
            cp1 = pltpu.async_copy(ts_h.at[sidx], sbuf, sem1)
            cp2 = pltpu.async_copy(td_h.at[didx], dbuf, sem2)
            cp1.wait()
            cp2.wait()

            def pk(r, carry2):
                for dcol, scol in packs:
                    sbuf[r, pl.ds(scol, 16)] = dbuf[r, pl.ds(dcol, 16)]
                return carry2

            lax.fori_loop(0, B, pk, 0)
            pltpu.sync_copy(sbuf, out_h.at[pl.ds(off, B)])
            return carry

        lax.fori_loop(0, iters, body, 0)

    return gk(srcg, dstg, ts, td)


def _segsum(rows, dsts, nr):
    return jax.ops.segment_sum(rows, dsts, num_segments=nr)


def _prep_body(x_ref, pos_ref, lin_ref, lsrc_ref, ldst_ref, ts_ref, td_ref):
    x = x_ref[...]
    pos = pos_ref[...]
    nb = x.shape[0]
    c = lin_ref.shape[1]
    xv = jnp.dot(x, lin_ref[...], preferred_element_type=jnp.float32)
    a_s = jnp.dot(x, lsrc_ref[...], preferred_element_type=jnp.float32)
    a_d = jnp.dot(x, ldst_ref[...], preferred_element_type=jnp.float32)
    pad_s = jnp.zeros((nb, TW - 2 * c - 2), jnp.float32)
    pad_d = jnp.zeros((nb, TW - c - 2), jnp.float32)
    ts_ref[...] = jnp.concatenate([xv, a_s, pos, pad_s], axis=1)
    td_ref[...] = jnp.concatenate([a_d, pos, pad_d], axis=1)


def _prep(x, pos, lin, lsrc, ldst, nb=400):
    n, cin = x.shape
    c = lin.shape[1]
    grid = n // nb
    row = lambda i: (i, 0)
    w0 = lambda i: (0, 0)
    return pl.pallas_call(
        _prep_body,
        grid=(grid,),
        in_specs=[
            pl.BlockSpec((nb, cin), row),
            pl.BlockSpec((nb, 2), row),
            pl.BlockSpec((cin, c), w0),
            pl.BlockSpec((cin, c), w0),
            pl.BlockSpec((cin, c), w0),
        ],
        out_specs=[
            pl.BlockSpec((nb, TW), row),
            pl.BlockSpec((nb, TW), row),
        ],
        out_shape=[
            jax.ShapeDtypeStruct((n, TW), jnp.float32),
            jax.ShapeDtypeStruct((n, TW), jnp.float32),
        ],
    )(x, pos, lin, lsrc, ldst)


def _mid_body(c, offs, sg_ref,
              pw1_ref, pb1_ref, pw2_ref, pb2_ref,
              aw1_ref, ab1_ref, aw2_ref, ab2_ref, out_ref):
    xo, ao, po, ado, pdo = offs
    sg = sg_ref[...]
    xv = sg[:, xo:xo + c]
    a_s = sg[:, ao:ao + c]
    pos_s = sg[:, po:po + 2]
    a_d = sg[:, ado:ado + c]
    pos_d = sg[:, pdo:pdo + 2]
    pd = pos_d - pos_s
    hp = jax.nn.relu(
        jnp.dot(pd, pw1_ref[...], preferred_element_type=jnp.float32)
        + pb1_ref[...])
    delta = jnp.dot(hp, pw2_ref[...], preferred_element_type=jnp.float32) + pb2_ref[...]
    apre = a_d - a_s + delta
    hid = jax.nn.relu(
        jnp.dot(apre, aw1_ref[...], preferred_element_type=jnp.float32)
        + ab1_ref[...])
    alpha = jnp.dot(hid, aw2_ref[...], preferred_element_type=jnp.float32) + ab2_ref[...]
    ex = jnp.exp(alpha)
    v = ex * (xv + delta)
    out_ref[...] = jnp.concatenate([v, ex], axis=1)


def _mid(c, offs, sg, pw1, pb1, pw2, pb2, aw1, ab1, aw2, ab2, be=2048):
    ep = sg.shape[0]
    sw = TW
    hidden = pw1.shape[1]
    grid = ep // be
    row = lambda i: (i, 0)
    w0 = lambda i: (0, 0)
    b0 = lambda i: (0,)
    return pl.pallas_call(
        functools.partial(_mid_body, c, offs),
        grid=(grid,),
        in_specs=[
            pl.BlockSpec((be, sw), row),
            pl.BlockSpec((2, hidden), w0), pl.BlockSpec((hidden,), b0),
            pl.BlockSpec((hidden, c), w0), pl.BlockSpec((c,), b0),
            pl.BlockSpec((c, hidden), w0), pl.BlockSpec((hidden,), b0),
            pl.BlockSpec((hidden, c), w0), pl.BlockSpec((c,), b0),
        ],
        out_specs=[pl.BlockSpec((be, 2 * c), row)],
        out_shape=[jax.ShapeDtypeStruct((ep, 2 * c), jnp.float32)],
    )(sg, pw1, pb1, pw2, pb2, aw1, ab1, aw2, ab2)[0]


def _selfloop_terms(a_d, a_s, xv, pw1b, pw2, pb2, aw1, ab1, aw2, ab2):
    hp0 = jax.nn.relu(pw1b)
    delta0 = jnp.dot(hp0.reshape(1, -1), pw2,
                     preferred_element_type=jnp.float32) + pb2
    apre = a_d - a_s + delta0
    hid = jax.nn.relu(
        jnp.dot(apre, aw1, preferred_element_type=jnp.float32) + ab1)
    alpha = jnp.dot(hid, aw2, preferred_element_type=jnp.float32) + ab2
    ex = jnp.exp(alpha)
    return ex, ex * (xv + delta0)


def _final1_body(c, acc_ref, ts_ref, td_ref,
                 pb1_ref, pw2_ref, pb2_ref, aw1_ref, ab1_ref, aw2_ref, ab2_ref,
                 lin2_ref, lsrc2_ref, ldst2_ref, t2_ref):
    num = acc_ref[:, 0:c]
    den = acc_ref[:, c:2 * c]
    sg = ts_ref[...]
    dg = td_ref[...]
    xv = sg[:, 0:c]
    a_s = sg[:, c:2 * c]
    pos = sg[:, 2 * c:2 * c + 2]
    a_d = dg[:, 0:c]
    ex_sl, v_sl = _selfloop_terms(
        a_d, a_s, xv, pb1_ref[...], pw2_ref[...], pb2_ref[...],
        aw1_ref[...], ab1_ref[...], aw2_ref[...], ab2_ref[...])
    hmat = jax.nn.relu((num + v_sl) / (den + ex_sl + 1e-16))
    nb = hmat.shape[0]
    c2 = lin2_ref.shape[1]
    xv2 = jnp.dot(hmat, lin2_ref[...], preferred_element_type=jnp.float32)
    a_s2 = jnp.dot(hmat, lsrc2_ref[...], preferred_element_type=jnp.float32)
    a_d2 = jnp.dot(hmat, ldst2_ref[...], preferred_element_type=jnp.float32)
    z6 = jnp.zeros((nb, 6), jnp.float32)
    zrest = jnp.zeros((nb, TW - 34), jnp.float32)
    t2_ref[...] = jnp.concatenate(
        [xv2, a_s2, pos, z6, a_d2, pos, zrest], axis=1)


def _final1(c, acc, ts, td, pb1, pw2, pb2, aw1, ab1, aw2, ab2,
            lin2, lsrc2, ldst2, nb=400):
    n = ts.shape[0]
    hidden = pw2.shape[0]
    c2 = lin2.shape[1]
    grid = n // nb
    row = lambda i: (i, 0)
    w0 = lambda i: (0, 0)
    b0 = lambda i: (0,)
    return pl.pallas_call(
        functools.partial(_final1_body, c),
        grid=(grid,),
        in_specs=[
            pl.BlockSpec((nb, 2 * c), row),
            pl.BlockSpec((nb, TW), row),
            pl.BlockSpec((nb, TW), row),
            pl.BlockSpec((hidden,), b0),
            pl.BlockSpec((hidden, c), w0), pl.BlockSpec((c,), b0),
            pl.BlockSpec((c, hidden), w0), pl.BlockSpec((hidden,), b0),
            pl.BlockSpec((hidden, c), w0), pl.BlockSpec((c,), b0),
            pl.BlockSpec((c, c2), w0), pl.BlockSpec((c, c2), w0),
            pl.BlockSpec((c, c2), w0),
        ],
        out_specs=[pl.BlockSpec((nb, TW), row)],
        out_shape=[jax.ShapeDtypeStruct((n, TW), jnp.float32)],
    )(acc, ts, td, pb1, pw2, pb2, aw1, ab1, aw2, ab2,
      lin2, lsrc2, ldst2)[0]


def _final2_body(c, acc_ref, t2_ref,
                 pb1_ref, pw2_ref, pb2_ref, aw1_ref, ab1_ref, aw2_ref, ab2_ref,
                 out_ref):
    num = acc_ref[:, 0:c]
    den = acc_ref[:, c:2 * c]
    t2 = t2_ref[...]
    xv = t2[:, 0:c]
    a_s = t2[:, c:2 * c]
    a_d = t2[:, 24:24 + c]
    ex_sl, v_sl = _selfloop_terms(
        a_d, a_s, xv, pb1_ref[...], pw2_ref[...], pb2_ref[...],
        aw1_ref[...], ab1_ref[...], aw2_ref[...], ab2_ref[...])
    out_ref[...] = (num + v_sl) / (den + ex_sl + 1e-16)


def _final2(c, acc, t2, pb1, pw2, pb2, aw1, ab1, aw2, ab2, nb=400):
    n = t2.shape[0]
    hidden = pw2.shape[0]
    grid = n // nb
    row = lambda i: (i, 0)
    w0 = lambda i: (0, 0)
    b0 = lambda i: (0,)
    return pl.pallas_call(
        functools.partial(_final2_body, c),
        grid=(grid,),
        in_specs=[
            pl.BlockSpec((nb, 2 * c), row),
            pl.BlockSpec((nb, TW), row),
            pl.BlockSpec((hidden,), b0),
            pl.BlockSpec((hidden, c), w0), pl.BlockSpec((c,), b0),
            pl.BlockSpec((c, hidden), w0), pl.BlockSpec((hidden,), b0),
            pl.BlockSpec((hidden, c), w0), pl.BlockSpec((c,), b0),
        ],
        out_specs=[pl.BlockSpec((nb, c), row)],
        out_shape=[jax.ShapeDtypeStruct((n, c), jnp.float32)],
    )(acc, t2, pb1, pw2, pb2, aw1, ab1, aw2, ab2)[0]


def kernel(x, pos, edge_index,
           l1_lin, l1_lsrc, l1_ldst, l1_pw1, l1_pb1, l1_pw2, l1_pb2,
           l1_aw1, l1_ab1, l1_aw2, l1_ab2,
           l2_lin, l2_lsrc, l2_ldst, l2_pw1, l2_pb1, l2_pw2, l2_pb2,
           l2_aw1, l2_ab1, l2_aw2, l2_ab2):
    n = x.shape[0]
    e = edge_index.shape[1]
    ep = _round_up(e, NW * B)
    pad = ep - e

    src = edge_index[0]
    dst = edge_index[1]
    zpad = jnp.zeros((pad,), jnp.int32)
    srcg = jnp.concatenate([src, zpad])
    dstg = jnp.concatenate([dst, zpad])
    dsts = jnp.concatenate([dst, jnp.full((pad,), n, jnp.int32)])

    c1, c2 = l1_lin.shape[1], l2_lin.shape[1]

    t1s, t1d = _prep(x, pos, l1_lin, l1_lsrc, l1_ldst)
    sg1 = _sc_gather(srcg, dstg, t1s, t1d, [(0, 80), (16, 96), (32, 112)])
    rows1 = _mid(c1, (0, 32, 64, 80, 112), sg1, l1_pw1, l1_pb1, l1_pw2,
                 l1_pb2, l1_aw1, l1_ab1, l1_aw2, l1_ab2)
    acc1 = _segsum(rows1, dsts, n)
    t2 = _final1(c1, acc1, t1s, t1d, l1_pb1, l1_pw2, l1_pb2,
                 l1_aw1, l1_ab1, l1_aw2, l1_ab2,
                 l2_lin, l2_lsrc, l2_ldst)

    sg2 = _sc_gather(srcg, dstg, t2, t2, [(24, 32)])
    rows2 = _mid(c2, (0, 8, 16, 32, 40), sg2, l2_pw1, l2_pb1, l2_pw2,
                 l2_pb2, l2_aw1, l2_ab1, l2_aw2, l2_ab2)
    acc2 = _segsum(rows2, dsts, n)
    out = _final2(c2, acc2, t2, l2_pb1, l2_pw2, l2_pb2,
                  l2_aw1, l2_ab1, l2_aw2, l2_ab2)
    return out
